# Initial kernel scaffold; baseline (speedup 1.0000x reference)
import jax
import jax.numpy as jnp
from jax.experimental import pallas as pl


def kernel(redisuals, edge_index):
    raise NotImplementedError("write your pallas kernel here")


# speedup vs baseline: 42.1974x; 42.1974x over previous
import functools

import jax
import jax.numpy as jnp
from jax import lax
from jax.experimental import pallas as pl
from jax.experimental.pallas import tpu as pltpu
from jax.experimental.pallas import tpu_sc as plsc

_N = 100000
_E = 3200000
_C = 16
_K = 10
_ALPHA = 0.9

_NC = 2
_NS = 16
_NW = _NC * _NS

_GRP = 128
_SCH = 8
_ROWS_PW = 784
_NSUP = _ROWS_PW // _SCH
_E_PAD = _NW * _ROWS_PW * _GRP
_NPAD = 100352
_RPT = _NPAD // _NS

_mesh = plsc.VectorSubcoreMesh(core_axis_name="c", subcore_axis_name="s")
_params = pltpu.CompilerParams(use_tc_tiling_on_sc=False)


@functools.partial(
    pl.kernel,
    out_type=jax.ShapeDtypeStruct((_NC, _NPAD, _C), jnp.float32),
    mesh=_mesh,
    scratch_types=[
        pltpu.VMEM_SHARED((_NPAD, _C), jnp.float32),
        pltpu.VMEM((_SCH, _GRP), jnp.int32),
        pltpu.VMEM((_SCH, _GRP), jnp.int32),
        pltpu.VMEM((_SCH, _GRP, _C), jnp.float32),
        pltpu.SemaphoreType.DMA,
    ],
    compiler_params=_params,
)
def _spmm(z_hbm, src_hbm, dst_hbm, zeros_hbm, out_hbm, acc, src_v, dst_v, rows_v, sem):
    cid = lax.axis_index("c")
    sid = lax.axis_index("s")
    wid = cid * _NS + sid
    pltpu.sync_copy(zeros_hbm.at[pl.ds(sid * _RPT, _RPT)],
                    acc.at[pl.ds(sid * _RPT, _RPT)])
    plsc.subcore_barrier()
    base = wid * _ROWS_PW

    @pl.loop(0, _NSUP)
    def _loop(i):
        row0 = base + i * _SCH
        pltpu.sync_copy(src_hbm.at[pl.ds(row0, _SCH)], src_v)
        pltpu.sync_copy(dst_hbm.at[pl.ds(row0, _SCH)], dst_v)
        for j in range(_SCH):
            pltpu.async_copy(z_hbm.at[src_v.at[j]], rows_v.at[j], sem)
        for j in range(_SCH):
            pltpu.make_async_copy(z_hbm.at[src_v.at[j]], rows_v.at[j], sem).wait()
        for j in range(_SCH):
            pltpu.sync_copy(rows_v.at[j], acc.at[dst_v.at[j]], add=True)

    plsc.subcore_barrier()
    pltpu.sync_copy(acc.at[pl.ds(sid * _RPT, _RPT)],
                    out_hbm.at[cid, pl.ds(sid * _RPT, _RPT)])


@functools.partial(
    pl.kernel,
    out_type=(jax.ShapeDtypeStruct((_NPAD,), jnp.float32),
              jax.ShapeDtypeStruct((_NPAD,), jnp.float32)),
    mesh=_mesh,
    scratch_types=[
        pltpu.VMEM_SHARED((_NPAD,), jnp.float32),
        pltpu.VMEM((_SCH, _GRP), jnp.int32),
        pltpu.VMEM((_GRP,), jnp.float32),
    ],
    compiler_params=_params,
)
def _deg(src_hbm, dst_hbm, zeros1_hbm, out0_hbm, out1_hbm, dacc, idx_v, ones_v):
    cid = lax.axis_index("c")
    sid = lax.axis_index("s")
    wid = cid * _NS + sid
    for t in range(_GRP // 16):
        ones_v[pl.ds(t * 16, 16)] = jnp.full((16,), 1.0, jnp.float32)
    pltpu.sync_copy(zeros1_hbm.at[pl.ds(sid * _RPT, _RPT)],
                    dacc.at[pl.ds(sid * _RPT, _RPT)])
    plsc.subcore_barrier()
    base = wid * _ROWS_PW

    @pl.loop(0, _NSUP)
    def _loop(i):
        row0 = base + i * _SCH
        pltpu.sync_copy(src_hbm.at[pl.ds(row0, _SCH)], idx_v)
        for j in range(_SCH):
            pltpu.sync_copy(ones_v, dacc.at[idx_v.at[j]], add=True)
        pltpu.sync_copy(dst_hbm.at[pl.ds(row0, _SCH)], idx_v)
        for j in range(_SCH):
            pltpu.sync_copy(ones_v, dacc.at[idx_v.at[j]], add=True)

    plsc.subcore_barrier()

    @pl.when(cid == 0)
    def _():
        pltpu.sync_copy(dacc.at[pl.ds(sid * _RPT, _RPT)],
                        out0_hbm.at[pl.ds(sid * _RPT, _RPT)])

    @pl.when(cid == 1)
    def _():
        pltpu.sync_copy(dacc.at[pl.ds(sid * _RPT, _RPT)],
                        out1_hbm.at[pl.ds(sid * _RPT, _RPT)])


def kernel(redisuals, edge_index):
    r = redisuals
    ei = edge_index.astype(jnp.int32)
    padv = jnp.full((_E_PAD - _E,), _N, jnp.int32)
    srcp = jnp.concatenate([ei[0], padv]).reshape(_E_PAD // _GRP, _GRP)
    dstp = jnp.concatenate([ei[1], padv]).reshape(_E_PAD // _GRP, _GRP)
    zeros2 = jnp.zeros((_NPAD, _C), jnp.float32)
    zeros1 = jnp.zeros((_NPAD,), jnp.float32)

    deg0, deg1 = _deg(srcp, dstp, zeros1)
    deg = deg0[:_N] + deg1[:_N]
    dis = jnp.where(deg > 0, lax.rsqrt(deg), 0.0)
    dinv = dis * dis

    zpad = jnp.zeros((_NPAD - _N, _C), jnp.float32)
    z = jnp.concatenate([dis[:, None] * r, zpad])
    s_a = jnp.concatenate([jnp.broadcast_to((_ALPHA * dinv)[:, None], (_N, _C)), zpad])
    b_a = jnp.concatenate([((1.0 - _ALPHA) * dis)[:, None] * r, zpad])
    s_f = jnp.concatenate([jnp.broadcast_to((_ALPHA * dis)[:, None], (_N, _C)), zpad])
    b_f = jnp.concatenate([(1.0 - _ALPHA) * r, zpad])

    for _ in range(_K - 1):
        p = _spmm(z, srcp, dstp, zeros2)
        z = s_a * (p[0] + p[1]) + b_a
    p = _spmm(z, srcp, dstp, zeros2)
    return (s_f * (p[0] + p[1]) + b_f)[:_N]


# speedup vs baseline: 55.9018x; 1.3248x over previous
import functools

import jax
import jax.numpy as jnp
from jax import lax
from jax.experimental import pallas as pl
from jax.experimental.pallas import tpu as pltpu
from jax.experimental.pallas import tpu_sc as plsc

_N = 100000
_E = 3200000
_C = 16
_K = 10
_ALPHA = 0.9

_NC = 2
_NS = 16
_NW = _NC * _NS

_GRP = 128
_SCH = 8
_SCS = 4
_ROWS_PW = 784
_NSUP = _ROWS_PW // _SCH
_NSUS = _ROWS_PW // _SCS
_E_PAD = _NW * _ROWS_PW * _GRP
_NPAD = 100352
_RPT = _NPAD // _NS
_RPW = _NPAD // _NW
_CCH = 784

_mesh = plsc.VectorSubcoreMesh(core_axis_name="c", subcore_axis_name="s")
_params = pltpu.CompilerParams(use_tc_tiling_on_sc=False)


@functools.partial(
    pl.kernel,
    out_type=jax.ShapeDtypeStruct((_NC, _NPAD, _C), jnp.float32),
    mesh=_mesh,
    scratch_types=[
        pltpu.VMEM_SHARED((_NPAD, _C), jnp.float32),
        pltpu.VMEM((_SCS, _GRP), jnp.int32),
        pltpu.VMEM((_SCS, _GRP), jnp.int32),
        pltpu.VMEM((_SCS, _GRP, _C), jnp.float32),
        pltpu.VMEM((_SCS, _GRP), jnp.int32),
        pltpu.VMEM((_SCS, _GRP), jnp.int32),
        pltpu.VMEM((_SCS, _GRP, _C), jnp.float32),
        pltpu.SemaphoreType.DMA,
        pltpu.SemaphoreType.DMA,
        pltpu.SemaphoreType.DMA,
    ],
    compiler_params=_params,
)
def _spmm(z_hbm, src_hbm, dst_hbm, zeros_hbm, out_hbm,
          acc, src_v0, dst_v0, rows_v0, src_v1, dst_v1, rows_v1,
          isem, gsem, ssem):
    cid = lax.axis_index("c")
    sid = lax.axis_index("s")
    wid = cid * _NS + sid
    pltpu.sync_copy(zeros_hbm.at[pl.ds(sid * _RPT, _RPT)],
                    acc.at[pl.ds(sid * _RPT, _RPT)])
    plsc.subcore_barrier()
    base = wid * _ROWS_PW

    bufs = ((src_v0, dst_v0, rows_v0), (src_v1, dst_v1, rows_v1))

    def chunk(row0, b, first):
        sv, dv, rv = bufs[b]
        if not first:
            for j in range(_SCS):
                pltpu.make_async_copy(rv.at[j], acc.at[dv.at[j]], ssem).wait()
        pltpu.async_copy(src_hbm.at[pl.ds(row0, _SCS)], sv, isem)
        pltpu.async_copy(dst_hbm.at[pl.ds(row0, _SCS)], dv, isem)
        pltpu.make_async_copy(src_hbm.at[pl.ds(row0, _SCS)], sv, isem).wait()
        pltpu.make_async_copy(dst_hbm.at[pl.ds(row0, _SCS)], dv, isem).wait()
        for j in range(_SCS):
            pltpu.async_copy(z_hbm.at[sv.at[j]], rv.at[j], gsem)
        for j in range(_SCS):
            pltpu.make_async_copy(z_hbm.at[sv.at[j]], rv.at[j], gsem).wait()
        for j in range(_SCS):
            pltpu.async_copy(rv.at[j], acc.at[dv.at[j]], ssem, add=True)

    chunk(base, 0, True)
    chunk(base + _SCS, 1, True)

    @pl.loop(0, (_NSUS - 2) // 2)
    def _loop(t):
        row0 = base + (2 * t + 2) * _SCS
        chunk(row0, 0, False)
        chunk(row0 + _SCS, 1, False)

    for b in range(2):
        sv, dv, rv = bufs[b]
        for j in range(_SCS):
            pltpu.make_async_copy(rv.at[j], acc.at[dv.at[j]], ssem).wait()
    plsc.subcore_barrier()
    pltpu.sync_copy(acc.at[pl.ds(sid * _RPT, _RPT)],
                    out_hbm.at[cid, pl.ds(sid * _RPT, _RPT)])


@functools.partial(
    pl.kernel,
    out_type=(jax.ShapeDtypeStruct((_NPAD,), jnp.float32),
              jax.ShapeDtypeStruct((_NPAD,), jnp.float32)),
    mesh=_mesh,
    scratch_types=[
        pltpu.VMEM_SHARED((_NPAD,), jnp.float32),
        pltpu.VMEM((_SCH, _GRP), jnp.int32),
        pltpu.VMEM((_SCH, _GRP), jnp.int32),
        pltpu.VMEM((_GRP,), jnp.float32),
        pltpu.SemaphoreType.DMA,
        pltpu.SemaphoreType.DMA,
    ],
    compiler_params=_params,
)
def _deg(src_hbm, dst_hbm, zeros1_hbm, out0_hbm, out1_hbm,
         dacc, idx_v0, idx_v1, ones_v, isem, ssem):
    cid = lax.axis_index("c")
    sid = lax.axis_index("s")
    wid = cid * _NS + sid
    for t in range(_GRP // 16):
        ones_v[pl.ds(t * 16, 16)] = jnp.full((16,), 1.0, jnp.float32)
    pltpu.sync_copy(zeros1_hbm.at[pl.ds(sid * _RPT, _RPT)],
                    dacc.at[pl.ds(sid * _RPT, _RPT)])
    plsc.subcore_barrier()
    base = wid * _ROWS_PW

    ibufs = (idx_v0, idx_v1)

    def chunk(idx_hbm, row0, b, first):
        iv = ibufs[b]
        if not first:
            for j in range(_SCH):
                pltpu.make_async_copy(ones_v, dacc.at[iv.at[j]], ssem).wait()
        pltpu.async_copy(idx_hbm.at[pl.ds(row0, _SCH)], iv, isem)
        pltpu.make_async_copy(idx_hbm.at[pl.ds(row0, _SCH)], iv, isem).wait()
        for j in range(_SCH):
            pltpu.async_copy(ones_v, dacc.at[iv.at[j]], ssem, add=True)

    chunk(src_hbm, base, 0, True)
    chunk(src_hbm, base + _SCH, 1, True)

    @pl.loop(0, (_NSUP - 2) // 2)
    def _loop_s(t):
        row0 = base + (2 * t + 2) * _SCH
        chunk(src_hbm, row0, 0, False)
        chunk(src_hbm, row0 + _SCH, 1, False)

    @pl.loop(0, _NSUP // 2)
    def _loop_d(t):
        row0 = base + 2 * t * _SCH
        chunk(dst_hbm, row0, 0, False)
        chunk(dst_hbm, row0 + _SCH, 1, False)

    for b in range(2):
        for j in range(_SCH):
            pltpu.make_async_copy(ones_v, dacc.at[ibufs[b].at[j]], ssem).wait()
    plsc.subcore_barrier()

    @pl.when(cid == 0)
    def _():
        pltpu.sync_copy(dacc.at[pl.ds(sid * _RPT, _RPT)],
                        out0_hbm.at[pl.ds(sid * _RPT, _RPT)])

    @pl.when(cid == 1)
    def _():
        pltpu.sync_copy(dacc.at[pl.ds(sid * _RPT, _RPT)],
                        out1_hbm.at[pl.ds(sid * _RPT, _RPT)])


@functools.partial(
    pl.kernel,
    out_type=jax.ShapeDtypeStruct((_NPAD, _C), jnp.float32),
    mesh=_mesh,
    scratch_types=[
        pltpu.VMEM((_CCH, _C), jnp.float32),
        pltpu.VMEM((_CCH, _C), jnp.float32),
        pltpu.VMEM((_CCH, _C), jnp.float32),
        pltpu.VMEM((_CCH, _C), jnp.float32),
        pltpu.VMEM((_CCH, _C), jnp.float32),
        pltpu.SemaphoreType.DMA,
    ],
    compiler_params=_params,
)
def _rescale(p_hbm, s_hbm, b_hbm, out_hbm, p0_v, p1_v, s_v, b_v, o_v, sem):
    cid = lax.axis_index("c")
    sid = lax.axis_index("s")
    wid = cid * _NS + sid
    base = wid * _RPW

    @pl.loop(0, _RPW // _CCH)
    def _loop(t):
        row0 = base + t * _CCH
        pltpu.async_copy(p_hbm.at[0, pl.ds(row0, _CCH)], p0_v, sem)
        pltpu.async_copy(p_hbm.at[1, pl.ds(row0, _CCH)], p1_v, sem)
        pltpu.async_copy(s_hbm.at[pl.ds(row0, _CCH)], s_v, sem)
        pltpu.async_copy(b_hbm.at[pl.ds(row0, _CCH)], b_v, sem)
        pltpu.make_async_copy(p_hbm.at[0, pl.ds(row0, _CCH)], p0_v, sem).wait()
        pltpu.make_async_copy(p_hbm.at[1, pl.ds(row0, _CCH)], p1_v, sem).wait()
        pltpu.make_async_copy(s_hbm.at[pl.ds(row0, _CCH)], s_v, sem).wait()
        pltpu.make_async_copy(b_hbm.at[pl.ds(row0, _CCH)], b_v, sem).wait()

        @pl.loop(0, _CCH, unroll=8)
        def _rows(r):
            o_v[r] = s_v[r] * (p0_v[r] + p1_v[r]) + b_v[r]

        pltpu.sync_copy(o_v, out_hbm.at[pl.ds(row0, _CCH)])


def kernel(redisuals, edge_index):
    r = redisuals
    ei = edge_index.astype(jnp.int32)
    padv = jnp.full((_E_PAD - _E,), _N, jnp.int32)
    srcp = jnp.concatenate([ei[0], padv]).reshape(_E_PAD // _GRP, _GRP)
    dstp = jnp.concatenate([ei[1], padv]).reshape(_E_PAD // _GRP, _GRP)
    zeros2 = jnp.zeros((_NPAD, _C), jnp.float32)
    zeros1 = jnp.zeros((_NPAD,), jnp.float32)

    deg0, deg1 = _deg(srcp, dstp, zeros1)
    deg = deg0[:_N] + deg1[:_N]
    dis = jnp.where(deg > 0, lax.rsqrt(deg), 0.0)
    dinv = dis * dis

    zpad = jnp.zeros((_NPAD - _N, _C), jnp.float32)
    z = jnp.concatenate([dis[:, None] * r, zpad])
    s_a = jnp.concatenate([jnp.broadcast_to((_ALPHA * dinv)[:, None], (_N, _C)), zpad])
    b_a = jnp.concatenate([((1.0 - _ALPHA) * dis)[:, None] * r, zpad])
    s_f = jnp.concatenate([jnp.broadcast_to((_ALPHA * dis)[:, None], (_N, _C)), zpad])
    b_f = jnp.concatenate([(1.0 - _ALPHA) * r, zpad])

    for _ in range(_K - 1):
        p = _spmm(z, srcp, dstp, zeros2)
        z = _rescale(p, s_a, b_a)
    p = _spmm(z, srcp, dstp, zeros2)
    return _rescale(p, s_f, b_f)[:_N]
